# baseline (speedup 1.0000x reference)
import numpy as np

import jax
import jax.numpy as jnp
from jax.experimental import pallas as pl
from jax.experimental.pallas import tpu as pltpu

_H1, _W1 = 10, 11
_P1 = _H1 * _W1
_H2, _W2 = 5, 6
_P2 = _H2 * _W2

_SELU_ALPHA = 1.6732632423543772
_SELU_SCALE = 1.0507009873554805


def _tap_coord(r, c, tap):
    if tap == 0:
        return r, c
    if tap == 1:
        return r - 1, c
    if tap == 2:
        return r + 1, c
    u = 1 if (c % 2) else 0
    if tap == 3:
        return r - 1 + u, c - 1
    if tap == 4:
        return r + u, c - 1
    if tap == 5:
        return r - 1 + u, c + 1
    return r + u, c + 1


def _build_S(h, w):
    p_tot = h * w
    S = np.zeros((7, p_tot, p_tot), np.float32)
    for r in range(h):
        for c in range(w):
            p = r * w + c
            for k in range(7):
                rr, cc = _tap_coord(r, c, k)
                if 0 <= rr < h and 0 <= cc < w:
                    S[k, rr * w + cc, p] = 1.0
    return S


def _build_poolT():
    out = np.zeros((_P1, _P2), np.float32)
    for i in range(_H2):
        for j in range(_W2):
            m = i * _W2 + j
            r, c = 2 * i + (j % 2), 2 * j
            for k in range(7):
                rr, cc = _tap_coord(r, c, k)
                if 0 <= rr < _H1 and 0 <= cc < _W1:
                    out[rr * _W1 + cc, m] = 1.0 / 7.0
    return out


_S1 = _build_S(_H1, _W1)
_S2 = _build_S(_H2, _W2)
_POOLT = _build_poolT()

_K_IDX2 = np.full((_P2, _P2), 7, np.int32)
for _k in range(7):
    _K_IDX2[_S2[_k] > 0] = _k


def _mk_conv_perco(w, s_flat32, p):
    cout, cin = w.shape[1], w.shape[2]
    wt = w.transpose(1, 2, 0).reshape(cout * cin, 7)
    m = jnp.matmul(wt, s_flat32)
    return m.reshape(cout, cin, p, p).reshape(cout, cin * p, p)


def _densify_body(src_ref, dst_ref):
    cout = src_ref.shape[0]
    p = src_ref.shape[2]
    for co in range(cout):
        dst_ref[:, co * p:(co + 1) * p] = src_ref[co]


def _densify(stack):
    cout, k, p = stack.shape
    return pl.pallas_call(
        _densify_body,
        out_shape=jax.ShapeDtypeStruct((k, cout * p), stack.dtype),
        compiler_params=pltpu.CompilerParams(
            vmem_limit_bytes=64 * 1024 * 1024),
    )(stack)


def _fused_body(x_ref, *refs):
    (m1_ref, m2_ref, m3_ref, m4_ref, m5_ref,
     pf_ref, m6_ref, m7_ref, bc_ref, b67_ref,
     w1_ref, w2_ref, w3_ref, ow_ref, fb_ref, out_ref, hcat_ref) = refs
    m_refs = [m1_ref, m2_ref, m3_ref, m4_ref, m5_ref]
    f32 = jnp.float32
    bf16 = jnp.bfloat16

    hcat_ref[:, 0:330] = x_ref[...]
    off = 330
    for l in range(5):
        acc = jnp.dot(hcat_ref[:, 0:off], m_refs[l][...],
                      preferred_element_type=f32) + bc_ref[l]
        hcat_ref[:, off:off + 1320] = jnp.maximum(acc, 0.0).astype(bf16)
        off += 1320

    s = jnp.dot(hcat_ref[...], pf_ref[...], preferred_element_type=f32)

    sb = s.astype(bf16)
    m6f = m6_ref[...]
    m6hi = m6f.astype(bf16)
    r1 = m6f - m6hi.astype(f32)
    m6lo = r1.astype(bf16)
    m6lo2 = (r1 - m6lo.astype(f32)).astype(bf16)
    h6acc = jnp.dot(sb, m6hi, preferred_element_type=f32) \
        + jnp.dot(sb, m6lo, preferred_element_type=f32) \
        + jnp.dot(sb, m6lo2, preferred_element_type=f32) + b67_ref[0]
    h6 = jnp.maximum(h6acc, 0.0).astype(bf16)
    h7 = jnp.maximum(jnp.dot(h6, m7_ref[...], preferred_element_type=f32)
                     + b67_ref[1], 0.0).astype(bf16)

    def selu(v):
        return _SELU_SCALE * jnp.where(
            v > 0, v, _SELU_ALPHA * (jnp.exp(jnp.minimum(v, 0.0)) - 1.0))

    h = selu(jnp.dot(h7, w1_ref[...], preferred_element_type=f32) + fb_ref[0])
    h = selu(jnp.dot(h.astype(bf16), w2_ref[...], preferred_element_type=f32)
             + fb_ref[1])
    h = selu(jnp.dot(h.astype(bf16), w3_ref[...], preferred_element_type=f32)
             + fb_ref[2])
    out_ref[...] = (jnp.sum(h * ow_ref[0], axis=-1, keepdims=True)
                    + fb_ref[3, 0])


def _const_spec(arr):
    nd = arr.ndim
    return pl.BlockSpec(arr.shape, lambda i, _nd=nd: (0,) * _nd)


def kernel(x, conv_w_0, conv_w_1, conv_w_2, conv_w_3, conv_w_4, conv_w_5,
           conv_w_6, conv_b, fc_w_0, fc_w_1, fc_w_2, out_w, fc_b):
    B = x.shape[0]
    xb = x.reshape(B, 3 * _P1).astype(jnp.bfloat16)

    s1_flat = jnp.asarray(_S1.reshape(7, _P1 * _P1), jnp.bfloat16)
    s2_flat = jnp.asarray(_S2.reshape(7, _P2 * _P2), jnp.bfloat16)

    conv_ws = [conv_w_0, conv_w_1, conv_w_2, conv_w_3, conv_w_4]
    m_layers = [_densify(_mk_conv_perco(w, s1_flat, _P1)) for w in conv_ws]

    pf = jnp.asarray(np.tile(_POOLT, (63, 1)), jnp.bfloat16)

    wsum6 = jnp.sum(conv_w_5.astype(jnp.float32), axis=2)
    wsum6p = jnp.concatenate([wsum6, jnp.zeros((1, 63), jnp.float32)])
    m6 = wsum6p[_K_IDX2].transpose(0, 2, 1).reshape(_P2, 63 * _P2)
    m7 = _densify(_mk_conv_perco(conv_w_6, s2_flat, _P2))

    bc = jnp.repeat(conv_b[:12, :5].T, _P1, axis=1)
    b67 = jnp.stack([jnp.repeat(conv_b[:, 5], _P2),
                     jnp.repeat(conv_b[:, 6], _P2)])

    tb = 512
    bp = ((B + tb - 1) // tb) * tb
    if bp != B:
        xb = jnp.pad(xb, ((0, bp - B), (0, 0)))

    consts = m_layers + [pf, m6, m7, bc, b67,
                         fc_w_0, fc_w_1, fc_w_2, out_w, fc_b]
    out = pl.pallas_call(
        _fused_body,
        out_shape=jax.ShapeDtypeStruct((bp, 1), jnp.float32),
        grid=(bp // tb,),
        in_specs=[pl.BlockSpec((tb, 3 * _P1), lambda i: (i, 0))]
                 + [_const_spec(a) for a in consts],
        out_specs=pl.BlockSpec((tb, 1), lambda i: (i, 0)),
        scratch_shapes=[pltpu.VMEM((tb, 63 * _P1), jnp.bfloat16)],
        compiler_params=pltpu.CompilerParams(
            dimension_semantics=("parallel",),
            vmem_limit_bytes=64 * 1024 * 1024,
        ),
    )(xb, *consts)
    return out[:B]


# speedup vs baseline: 1.0709x; 1.0709x over previous
import numpy as np

import jax
import jax.numpy as jnp
from jax.experimental import pallas as pl
from jax.experimental.pallas import tpu as pltpu

_H1, _W1 = 10, 11
_P1 = _H1 * _W1
_H2, _W2 = 5, 6
_P2 = _H2 * _W2

_SELU_ALPHA = 1.6732632423543772
_SELU_SCALE = 1.0507009873554805


def _tap_coord(r, c, tap):
    if tap == 0:
        return r, c
    if tap == 1:
        return r - 1, c
    if tap == 2:
        return r + 1, c
    u = 1 if (c % 2) else 0
    if tap == 3:
        return r - 1 + u, c - 1
    if tap == 4:
        return r + u, c - 1
    if tap == 5:
        return r - 1 + u, c + 1
    return r + u, c + 1


def _build_S(h, w):
    p_tot = h * w
    S = np.zeros((7, p_tot, p_tot), np.float32)
    for r in range(h):
        for c in range(w):
            p = r * w + c
            for k in range(7):
                rr, cc = _tap_coord(r, c, k)
                if 0 <= rr < h and 0 <= cc < w:
                    S[k, rr * w + cc, p] = 1.0
    return S


def _build_poolT():
    out = np.zeros((_P1, _P2), np.float32)
    for i in range(_H2):
        for j in range(_W2):
            m = i * _W2 + j
            r, c = 2 * i + (j % 2), 2 * j
            for k in range(7):
                rr, cc = _tap_coord(r, c, k)
                if 0 <= rr < _H1 and 0 <= cc < _W1:
                    out[rr * _W1 + cc, m] = 1.0 / 7.0
    return out


_S1 = _build_S(_H1, _W1)
_S2 = _build_S(_H2, _W2)
_POOLT = _build_poolT()

_K_IDX2 = np.full((_P2, _P2), 7, np.int32)
for _k in range(7):
    _K_IDX2[_S2[_k] > 0] = _k


def _mk_conv_perco(w, s_flat32, p):
    cout, cin = w.shape[1], w.shape[2]
    wt = w.transpose(1, 2, 0).reshape(cout * cin, 7)
    m = jnp.matmul(wt, s_flat32)
    return m.reshape(cout, cin, p, p).reshape(cout, cin * p, p)


def _densify_body(src_ref, dst_ref):
    cout = src_ref.shape[0]
    p = src_ref.shape[2]
    for co in range(cout):
        dst_ref[:, co * p:(co + 1) * p] = src_ref[co]


def _densify(stack):
    cout, k, p = stack.shape
    return pl.pallas_call(
        _densify_body,
        out_shape=jax.ShapeDtypeStruct((k, cout * p), stack.dtype),
        compiler_params=pltpu.CompilerParams(
            vmem_limit_bytes=64 * 1024 * 1024),
    )(stack)


def _fused_body(x_ref, *refs):
    (m1_ref, m2_ref, m3_ref, m4_ref, m5_ref,
     pf_ref, m6_ref, m7_ref, bc_ref, b67_ref,
     w1_ref, w2_ref, w3_ref, ow_ref, fb_ref, out_ref, hcat_ref) = refs
    m_refs = [m1_ref, m2_ref, m3_ref, m4_ref, m5_ref]
    f32 = jnp.float32
    bf16 = jnp.bfloat16

    hcat_ref[:, 0:330] = x_ref[...]
    off = 330
    for l in range(5):
        acc = jnp.dot(hcat_ref[:, 0:off], m_refs[l][...],
                      preferred_element_type=f32) + bc_ref[l]
        hcat_ref[:, off:off + 1320] = jnp.maximum(acc, 0.0).astype(bf16)
        off += 1320

    s = jnp.dot(hcat_ref[...], pf_ref[...], preferred_element_type=f32)

    sb = s.astype(bf16)
    m6f = m6_ref[...]
    m6hi = m6f.astype(bf16)
    r1 = m6f - m6hi.astype(f32)
    m6lo = r1.astype(bf16)
    m6lo2 = (r1 - m6lo.astype(f32)).astype(bf16)
    h6acc = jnp.dot(sb, m6hi, preferred_element_type=f32) \
        + jnp.dot(sb, m6lo, preferred_element_type=f32) \
        + jnp.dot(sb, m6lo2, preferred_element_type=f32) + b67_ref[0]
    h6 = jnp.maximum(h6acc, 0.0).astype(bf16)
    h7 = jnp.maximum(jnp.dot(h6, m7_ref[...], preferred_element_type=f32)
                     + b67_ref[1], 0.0).astype(bf16)

    def selu(v):
        return _SELU_SCALE * jnp.where(
            v > 0, v, _SELU_ALPHA * (jnp.exp(jnp.minimum(v, 0.0)) - 1.0))

    h = selu(jnp.dot(h7, w1_ref[...], preferred_element_type=f32) + fb_ref[0])
    h = selu(jnp.dot(h.astype(bf16), w2_ref[...], preferred_element_type=f32)
             + fb_ref[1])
    h = selu(jnp.dot(h.astype(bf16), w3_ref[...], preferred_element_type=f32)
             + fb_ref[2])
    out_ref[...] = (jnp.sum(h * ow_ref[0], axis=-1, keepdims=True)
                    + fb_ref[3, 0])


def _const_spec(arr):
    nd = arr.ndim
    return pl.BlockSpec(arr.shape, lambda i, _nd=nd: (0,) * _nd)


def kernel(x, conv_w_0, conv_w_1, conv_w_2, conv_w_3, conv_w_4, conv_w_5,
           conv_w_6, conv_b, fc_w_0, fc_w_1, fc_w_2, out_w, fc_b):
    B = x.shape[0]
    xb = x.reshape(B, 3 * _P1).astype(jnp.bfloat16)

    s1_flat = jnp.asarray(_S1.reshape(7, _P1 * _P1), jnp.bfloat16)
    s2_flat = jnp.asarray(_S2.reshape(7, _P2 * _P2), jnp.bfloat16)

    conv_ws = [conv_w_0, conv_w_1, conv_w_2, conv_w_3, conv_w_4]
    m_layers = [_densify(_mk_conv_perco(w, s1_flat, _P1)) for w in conv_ws]

    pf = jnp.asarray(np.tile(_POOLT, (63, 1)), jnp.bfloat16)

    wsum6 = jnp.sum(conv_w_5.astype(jnp.float32), axis=2)
    wsum6p = jnp.concatenate([wsum6, jnp.zeros((1, 63), jnp.float32)])
    m6 = wsum6p[_K_IDX2].transpose(0, 2, 1).reshape(_P2, 63 * _P2)
    m7 = _densify(_mk_conv_perco(conv_w_6, s2_flat, _P2))

    bc = jnp.repeat(conv_b[:12, :5].T, _P1, axis=1)
    b67 = jnp.stack([jnp.repeat(conv_b[:, 5], _P2),
                     jnp.repeat(conv_b[:, 6], _P2)])

    tb = 256
    bp = ((B + tb - 1) // tb) * tb
    if bp != B:
        xb = jnp.pad(xb, ((0, bp - B), (0, 0)))

    consts = m_layers + [pf, m6, m7, bc, b67,
                         fc_w_0, fc_w_1, fc_w_2, out_w, fc_b]
    out = pl.pallas_call(
        _fused_body,
        out_shape=jax.ShapeDtypeStruct((bp, 1), jnp.float32),
        grid=(bp // tb,),
        in_specs=[pl.BlockSpec((tb, 3 * _P1), lambda i: (i, 0))]
                 + [_const_spec(a) for a in consts],
        out_specs=pl.BlockSpec((tb, 1), lambda i: (i, 0)),
        scratch_shapes=[pltpu.VMEM((tb, 63 * _P1), jnp.bfloat16)],
        compiler_params=pltpu.CompilerParams(
            dimension_semantics=("parallel",),
            vmem_limit_bytes=64 * 1024 * 1024,
        ),
    )(xb, *consts)
    return out[:B]


# speedup vs baseline: 1.0720x; 1.0011x over previous
import numpy as np

import jax
import jax.numpy as jnp
from jax.experimental import pallas as pl
from jax.experimental.pallas import tpu as pltpu

_H1, _W1 = 10, 11
_P1 = _H1 * _W1
_H2, _W2 = 5, 6
_P2 = _H2 * _W2

_SELU_ALPHA = 1.6732632423543772
_SELU_SCALE = 1.0507009873554805


def _tap_coord(r, c, tap):
    if tap == 0:
        return r, c
    if tap == 1:
        return r - 1, c
    if tap == 2:
        return r + 1, c
    u = 1 if (c % 2) else 0
    if tap == 3:
        return r - 1 + u, c - 1
    if tap == 4:
        return r + u, c - 1
    if tap == 5:
        return r - 1 + u, c + 1
    return r + u, c + 1


def _build_S(h, w):
    p_tot = h * w
    S = np.zeros((7, p_tot, p_tot), np.float32)
    for r in range(h):
        for c in range(w):
            p = r * w + c
            for k in range(7):
                rr, cc = _tap_coord(r, c, k)
                if 0 <= rr < h and 0 <= cc < w:
                    S[k, rr * w + cc, p] = 1.0
    return S


def _build_poolT():
    out = np.zeros((_P1, _P2), np.float32)
    for i in range(_H2):
        for j in range(_W2):
            m = i * _W2 + j
            r, c = 2 * i + (j % 2), 2 * j
            for k in range(7):
                rr, cc = _tap_coord(r, c, k)
                if 0 <= rr < _H1 and 0 <= cc < _W1:
                    out[rr * _W1 + cc, m] = 1.0 / 7.0
    return out


_S1 = _build_S(_H1, _W1)
_S2 = _build_S(_H2, _W2)
_POOLT = _build_poolT()

_K_IDX2 = np.full((_P2, _P2), 7, np.int32)
for _k in range(7):
    _K_IDX2[_S2[_k] > 0] = _k


def _mk_conv_perco(w, s_flat, p):
    cout, cin = w.shape[1], w.shape[2]
    wt = w.transpose(1, 2, 0).reshape(cout * cin, 7)
    m = jnp.matmul(wt, s_flat)
    return m.reshape(cout, cin, p, p).reshape(cout, cin * p, p)


def _densify_body(src_ref, dst_ref):
    cout = src_ref.shape[0]
    p = src_ref.shape[2]
    for co in range(cout):
        dst_ref[:, co * p:(co + 1) * p] = src_ref[co]


def _densify(stack):
    cout, k, p = stack.shape
    return pl.pallas_call(
        _densify_body,
        out_shape=jax.ShapeDtypeStruct((k, cout * p), stack.dtype),
        compiler_params=pltpu.CompilerParams(
            vmem_limit_bytes=64 * 1024 * 1024),
    )(stack)


def _fused_body(x_ref, *refs):
    (m1_ref, m2_ref, m3_ref, m4_ref, m5_ref,
     pf_ref, m6_ref, m7_ref, bc_ref, b67_ref,
     w1_ref, w2_ref, w3_ref, ow_ref, fb_ref, out_ref, hcat_ref) = refs
    m_refs = [m1_ref, m2_ref, m3_ref, m4_ref, m5_ref]
    f32 = jnp.float32
    bf16 = jnp.bfloat16

    hcat_ref[:, 0:330] = x_ref[...]
    off = 330
    for l in range(5):
        acc = jnp.dot(hcat_ref[:, 0:off], m_refs[l][...],
                      preferred_element_type=f32) + bc_ref[l]
        hcat_ref[:, off:off + 1320] = jnp.maximum(acc, 0.0).astype(bf16)
        off += 1320

    s = jnp.dot(hcat_ref[...], pf_ref[...], preferred_element_type=f32)

    sb = s.astype(bf16)
    m6f = m6_ref[...]
    m6hi = m6f.astype(bf16)
    r1 = m6f - m6hi.astype(f32)
    m6lo = r1.astype(bf16)
    m6lo2 = (r1 - m6lo.astype(f32)).astype(bf16)
    h6acc = jnp.dot(sb, m6hi, preferred_element_type=f32) \
        + jnp.dot(sb, m6lo, preferred_element_type=f32) \
        + jnp.dot(sb, m6lo2, preferred_element_type=f32) + b67_ref[0]
    h6 = jnp.maximum(h6acc, 0.0).astype(bf16)
    h7 = jnp.maximum(jnp.dot(h6, m7_ref[...], preferred_element_type=f32)
                     + b67_ref[1], 0.0).astype(bf16)

    def selu(v):
        return _SELU_SCALE * jnp.where(
            v > 0, v, _SELU_ALPHA * (jnp.exp(jnp.minimum(v, 0.0)) - 1.0))

    h = selu(jnp.dot(h7, w1_ref[...], preferred_element_type=f32) + fb_ref[0])
    h = selu(jnp.dot(h.astype(bf16), w2_ref[...], preferred_element_type=f32)
             + fb_ref[1])
    h = selu(jnp.dot(h.astype(bf16), w3_ref[...], preferred_element_type=f32)
             + fb_ref[2])
    out_ref[...] = (jnp.sum(h * ow_ref[0], axis=-1, keepdims=True)
                    + fb_ref[3, 0])


def _const_spec(arr):
    nd = arr.ndim
    return pl.BlockSpec(arr.shape, lambda i, _nd=nd: (0,) * _nd)


def kernel(x, conv_w_0, conv_w_1, conv_w_2, conv_w_3, conv_w_4, conv_w_5,
           conv_w_6, conv_b, fc_w_0, fc_w_1, fc_w_2, out_w, fc_b):
    B = x.shape[0]
    xb = x.reshape(B, 3 * _P1).astype(jnp.bfloat16)

    s1_flat = jnp.asarray(_S1.reshape(7, _P1 * _P1), jnp.bfloat16)
    s2_flat = jnp.asarray(_S2.reshape(7, _P2 * _P2), jnp.bfloat16)

    conv_ws = [conv_w_0, conv_w_1, conv_w_2, conv_w_3, conv_w_4]
    m_layers = [_densify(_mk_conv_perco(w, s1_flat, _P1)) for w in conv_ws]

    pf = jnp.asarray(np.tile(_POOLT, (63, 1)), jnp.bfloat16)

    wsum6 = jnp.sum(conv_w_5.astype(jnp.float32), axis=2)
    wsum6p = jnp.concatenate([wsum6, jnp.zeros((1, 63), jnp.float32)])
    m6 = wsum6p[_K_IDX2].transpose(0, 2, 1).reshape(_P2, 63 * _P2)
    m7 = _densify(_mk_conv_perco(conv_w_6, s2_flat, _P2))

    bc = jnp.repeat(conv_b[:12, :5].T, _P1, axis=1)
    b67 = jnp.stack([jnp.repeat(conv_b[:, 5], _P2),
                     jnp.repeat(conv_b[:, 6], _P2)])

    tb = 256
    bp = ((B + tb - 1) // tb) * tb
    if bp != B:
        xb = jnp.pad(xb, ((0, bp - B), (0, 0)))

    consts = m_layers + [pf, m6, m7, bc, b67,
                         fc_w_0, fc_w_1, fc_w_2, out_w, fc_b]
    out = pl.pallas_call(
        _fused_body,
        out_shape=jax.ShapeDtypeStruct((bp, 1), jnp.float32),
        grid=(bp // tb,),
        in_specs=[pl.BlockSpec((tb, 3 * _P1), lambda i: (i, 0))]
                 + [_const_spec(a) for a in consts],
        out_specs=pl.BlockSpec((tb, 1), lambda i: (i, 0)),
        scratch_shapes=[pltpu.VMEM((tb, 63 * _P1), jnp.bfloat16)],
        compiler_params=pltpu.CompilerParams(
            dimension_semantics=("parallel",),
            vmem_limit_bytes=64 * 1024 * 1024,
        ),
    )(xb, *consts)
    return out[:B]
